# baseline (speedup 1.0000x reference)
import functools

import jax
import jax.numpy as jnp
from jax import lax
from jax.experimental import pallas as pl
from jax.experimental.pallas import tpu as pltpu

EMB = 64
ROWS = 2000


def _ln_rows(x, w, b, eps=1e-5):
    m = jnp.mean(x, axis=-1, keepdims=True)
    v = jnp.mean((x - m) ** 2, axis=-1, keepdims=True)
    return (x - m) / jnp.sqrt(v + eps) * w + b


def _embed_body(cf, clnw, clnb, cw1, cb1, cw2, cb2,
                vf, vlnw, vlnb, vw1, vb1, vw2, vb2, co, vo):
    c = _ln_rows(cf[...], clnw[...], clnb[...])
    c = jax.nn.relu(jnp.dot(c, cw1[...], preferred_element_type=jnp.float32, precision=jax.lax.Precision.HIGHEST) + cb1[...])
    co[...] = jax.nn.relu(jnp.dot(c, cw2[...], preferred_element_type=jnp.float32, precision=jax.lax.Precision.HIGHEST) + cb2[...])
    v = _ln_rows(vf[...], vlnw[...], vlnb[...])
    v = jax.nn.relu(jnp.dot(v, vw1[...], preferred_element_type=jnp.float32, precision=jax.lax.Precision.HIGHEST) + vb1[...])
    vo[...] = jax.nn.relu(jnp.dot(v, vw2[...], preferred_element_type=jnp.float32, precision=jax.lax.Precision.HIGHEST) + vb2[...])


def _embed(cf, vf, p):
    n = cf.shape[0]
    grid = n // ROWS
    row = lambda c: pl.BlockSpec((ROWS, c), lambda i: (i, 0))
    full = lambda r, c: pl.BlockSpec((r, c), lambda i: (0, 0))
    args = [
        cf, p['cons_ln_w'].reshape(1, 4), p['cons_ln_b'].reshape(1, 4),
        p['cons_W1'], p['cons_b1'].reshape(1, EMB),
        p['cons_W2'], p['cons_b2'].reshape(1, EMB),
        vf, p['var_ln_w'].reshape(1, 6), p['var_ln_b'].reshape(1, 6),
        p['var_W1'], p['var_b1'].reshape(1, EMB),
        p['var_W2'], p['var_b2'].reshape(1, EMB),
    ]
    in_specs = [
        row(4), full(1, 4), full(1, 4), full(4, EMB), full(1, EMB),
        full(EMB, EMB), full(1, EMB),
        row(6), full(1, 6), full(1, 6), full(6, EMB), full(1, EMB),
        full(EMB, EMB), full(1, EMB),
    ]
    return pl.pallas_call(
        _embed_body,
        grid=(grid,),
        in_specs=in_specs,
        out_specs=[row(EMB), row(EMB)],
        out_shape=[jax.ShapeDtypeStruct((n, EMB), jnp.float32)] * 2,
    )(*args)


def _pre_body(right, left, wl, blc, wr, a2o, bo):
    a2o[...] = jnp.dot(right[...], wl[...], preferred_element_type=jnp.float32, precision=jax.lax.Precision.HIGHEST) + blc[...]
    bo[...] = jnp.dot(left[...], wr[...], preferred_element_type=jnp.float32, precision=jax.lax.Precision.HIGHEST)


def _pre(right, left, wl, blc, wr):
    n = right.shape[0]
    grid = n // ROWS
    row = pl.BlockSpec((ROWS, EMB), lambda i: (i, 0))
    full = lambda r, c: pl.BlockSpec((r, c), lambda i: (0, 0))
    return pl.pallas_call(
        _pre_body,
        grid=(grid,),
        in_specs=[row, row, full(EMB, EMB), full(1, EMB), full(EMB, EMB)],
        out_specs=[row, row],
        out_shape=[jax.ShapeDtypeStruct((n, EMB), jnp.float32)] * 2,
    )(right, left, wl, blc, wr)


def _post_body(s, deg, right, wf, bf, lnw, lnb, wo1a, wo1b, bo1, wo2, bo2, out):
    agg = jnp.dot(s[...], wf[...], preferred_element_type=jnp.float32, precision=jax.lax.Precision.HIGHEST) + deg[...] * bf[...]
    agg = _ln_rows(agg, lnw[...], lnb[...])
    h = jax.nn.relu(jnp.dot(agg, wo1a[...], preferred_element_type=jnp.float32, precision=jax.lax.Precision.HIGHEST)
                    + jnp.dot(right[...], wo1b[...], preferred_element_type=jnp.float32, precision=jax.lax.Precision.HIGHEST)
                    + bo1[...])
    out[...] = jnp.dot(h, wo2[...], preferred_element_type=jnp.float32, precision=jax.lax.Precision.HIGHEST) + bo2[...]


def _post(s, deg, right, p):
    n = s.shape[0]
    grid = n // ROWS
    row = pl.BlockSpec((ROWS, EMB), lambda i: (i, 0))
    col = pl.BlockSpec((ROWS, 1), lambda i: (i, 0))
    full = lambda r, c: pl.BlockSpec((r, c), lambda i: (0, 0))
    return pl.pallas_call(
        _post_body,
        grid=(grid,),
        in_specs=[row, col, row, full(EMB, EMB), full(1, EMB), full(1, EMB),
                  full(1, EMB), full(EMB, EMB), full(EMB, EMB), full(1, EMB),
                  full(EMB, EMB), full(1, EMB)],
        out_specs=row,
        out_shape=jax.ShapeDtypeStruct((n, EMB), jnp.float32),
    )(s, deg.reshape(n, 1), right, p['Wf'], p['bf'].reshape(1, EMB),
      p['ln_pc_w'].reshape(1, EMB), p['ln_pc_b'].reshape(1, EMB),
      p['Wo1'][:EMB], p['Wo1'][EMB:], p['bo1'].reshape(1, EMB),
      p['Wo2'], p['bo2'].reshape(1, EMB))


def _head_body(v, w1, b1, w2, out):
    h = jax.nn.relu(jnp.dot(v[...], w1[...], preferred_element_type=jnp.float32, precision=jax.lax.Precision.HIGHEST) + b1[...])
    out[...] = jnp.dot(h, w2[...], preferred_element_type=jnp.float32, precision=jax.lax.Precision.HIGHEST)


def _head(v, p):
    n = v.shape[0]
    grid = n // ROWS
    row = pl.BlockSpec((ROWS, EMB), lambda i: (i, 0))
    full = lambda r, c: pl.BlockSpec((r, c), lambda i: (0, 0))
    out = pl.pallas_call(
        _head_body,
        grid=(grid,),
        in_specs=[row, full(EMB, EMB), full(1, EMB), full(EMB, 1)],
        out_specs=pl.BlockSpec((ROWS, 1), lambda i: (i, 0)),
        out_shape=jax.ShapeDtypeStruct((n, 1), jnp.float32),
    )(v, p['out_W1'], p['out_b1'].reshape(1, EMB), p['out_W2'])
    return out[:, 0]


def _edge_stage_ln(a2, b, tgt, src, n_out, lnw, lnb):
    m = a2[tgt] + b[src]
    t = jax.nn.relu(_ln_rows(m, lnw, lnb))
    return jnp.zeros((n_out, EMB), jnp.float32).at[tgt].add(t)


def kernel(constraint_features, edge_indices, edge_features, variable_features, params):
    del edge_features
    p = params
    n_cons = constraint_features.shape[0]
    n_vars = variable_features.shape[0]
    ei0, ei1 = edge_indices[0], edge_indices[1]

    c, v = _embed(constraint_features, variable_features, p)

    deg_c = jnp.zeros((n_cons,), jnp.float32).at[ei0].add(1.0)
    deg_v = jnp.zeros((n_vars,), jnp.float32).at[ei1].add(1.0)

    for name, left_is_v in (('c1', True), ('c2', False), ('c3', True), ('c4', False)):
        cp = p[name]
        left, right = (v, c) if left_is_v else (c, v)
        tgt, src = (ei0, ei1) if left_is_v else (ei1, ei0)
        deg = deg_c if left_is_v else deg_v
        blc = (cp['bl'] + cp['We'][0] * p['edge_ln_b'][0]).reshape(1, EMB)
        a2, b = _pre(right, left, cp['Wl'], blc, cp['Wr'])
        s = _edge_stage_ln(a2, b, tgt, src, right.shape[0],
                           cp['ln_f_w'], cp['ln_f_b'])
        new_right = _post(s, deg, right, cp)
        if left_is_v:
            c = new_right
        else:
            v = new_right

    return _head(v, p)
